# baseline (speedup 1.0000x reference)
import functools

import jax
import jax.numpy as jnp
from jax import lax
from jax.experimental import pallas as pl
from jax.experimental.pallas import tpu as pltpu
from jax.experimental.pallas import tpu_sc as plsc

_L = 16


def _make_embed_kernel(B, S, D, n_workers, chunk, nbuf):
    pos_per_w = S // n_workers
    halves = pos_per_w // chunk
    n_chunks = B * halves

    mesh = plsc.VectorSubcoreMesh(core_axis_name="c", subcore_axis_name="s")

    @functools.partial(
        pl.kernel,
        mesh=mesh,
        out_type=jax.ShapeDtypeStruct((B * S, D), jnp.float32),
        scratch_types=[
            pltpu.VMEM((B * pos_per_w,), jnp.int32),
            pltpu.VMEM((chunk, D), jnp.float32),
        ]
        + [pltpu.VMEM((chunk, D), jnp.float32) for _ in range(nbuf)]
        + [
            pltpu.SemaphoreType.DMA,
            pltpu.SemaphoreType.DMA,
            pltpu.SemaphoreType.DMA,
        ],
    )
    def embed(idx_hbm, tok_hbm, pos_hbm, out_hbm, idx_v, pos_v, *rest):
        bufs, (gsem, wsem, psem) = rest[:nbuf], rest[nbuf:]
        nc = 2
        wid = lax.axis_index("s") * nc + lax.axis_index("c")
        pos_base = wid * pos_per_w

        def idx_copy(b):
            return pltpu.make_async_copy(
                idx_hbm.at[b, pl.ds(pos_base, pos_per_w)],
                idx_v.at[pl.ds(b * pos_per_w, pos_per_w)],
                psem,
            )

        for b in range(B):
            idx_copy(b).start()
        for b in range(B):
            idx_copy(b).wait()

        def idx_slice(ci):
            h, b = divmod(ci, B)
            return idx_v.at[pl.ds(b * pos_per_w + h * chunk, chunk)]

        def out_slice(ci):
            h, b = divmod(ci, B)
            return out_hbm.at[pl.ds(b * S + pos_base + h * chunk, chunk)]

        def pos_slice(h):
            return pos_hbm.at[pl.ds(pos_base + h * chunk, chunk)]

        ahead = nbuf - 2
        for ci in range(ahead):
            pltpu.async_copy(tok_hbm.at[idx_slice(ci)], bufs[ci], gsem)
        pltpu.async_copy(pos_slice(0), pos_v, psem)

        for ci in range(n_chunks):
            h, b = divmod(ci, B)
            buf = bufs[ci % nbuf]
            pltpu.make_async_copy(tok_hbm.at[idx_slice(ci)], buf, gsem).wait()
            if ci + ahead < n_chunks:
                if ci >= 2:
                    pltpu.make_async_copy(
                        bufs[(ci + ahead) % nbuf], out_slice(ci - 2), wsem
                    ).wait()
                pltpu.async_copy(
                    tok_hbm.at[idx_slice(ci + ahead)],
                    bufs[(ci + ahead) % nbuf],
                    gsem,
                )
            if b == 0:
                pltpu.make_async_copy(pos_slice(h), pos_v, psem).wait()

            def add_row(r, _, buf=buf):
                for c in range(D // _L):
                    sl = pl.ds(c * _L, _L)
                    plsc.addupdate(buf.at[r, sl], pos_v[r, sl])
                return 0

            lax.fori_loop(0, chunk, add_row, 0)
            if b == B - 1 and h + 1 < halves:
                pltpu.async_copy(pos_slice(h + 1), pos_v, psem)
            pltpu.async_copy(buf, out_slice(ci), wsem)

        for ci in range(max(0, n_chunks - nbuf), n_chunks):
            pltpu.make_async_copy(bufs[ci % nbuf], out_slice(ci), wsem).wait()

    return embed


def kernel(inputs, token_table, pos_table):
    B, S = inputs.shape
    V, D = token_table.shape
    idx2d = inputs.astype(jnp.int32)
    embed = _make_embed_kernel(B, S, D, n_workers=32, chunk=16, nbuf=8)
    out = embed(idx2d, token_table, pos_table)
    return out.reshape(B, S, D)


# speedup vs baseline: 1.1592x; 1.1592x over previous
import functools

import jax
import jax.numpy as jnp
from jax import lax
from jax.experimental import pallas as pl
from jax.experimental.pallas import tpu as pltpu
from jax.experimental.pallas import tpu_sc as plsc

_L = 16


def _make_embed_kernel(B, S, D, n_workers, chunk, nbuf):
    pos_per_w = S // n_workers
    halves = pos_per_w // chunk
    n_chunks = B * halves

    mesh = plsc.VectorSubcoreMesh(core_axis_name="c", subcore_axis_name="s")

    @functools.partial(
        pl.kernel,
        mesh=mesh,
        out_type=jax.ShapeDtypeStruct((B * S, D), jnp.float32),
        scratch_types=[
            pltpu.VMEM((B * pos_per_w,), jnp.int32),
            pltpu.VMEM((chunk, D), jnp.float32),
        ]
        + [pltpu.VMEM((chunk, D), jnp.float32) for _ in range(nbuf)]
        + [
            pltpu.SemaphoreType.DMA,
            pltpu.SemaphoreType.DMA,
            pltpu.SemaphoreType.DMA,
            pltpu.SemaphoreType.DMA,
        ],
    )
    def embed(idx_hbm, tok_hbm, pos_hbm, out_hbm, idx_v, pos_v, *rest):
        bufs, (gsem, wsem, psem, isem) = rest[:nbuf], rest[nbuf:]
        nc = 2
        wid = lax.axis_index("s") * nc + lax.axis_index("c")
        pos_base = wid * pos_per_w

        def idx_copy(b):
            return pltpu.make_async_copy(
                idx_hbm.at[b, pl.ds(pos_base, pos_per_w)],
                idx_v.at[pl.ds(b * pos_per_w, pos_per_w)],
                isem,
            )

        def idx_slice(ci):
            h, b = divmod(ci, B)
            return idx_v.at[pl.ds(b * pos_per_w + h * chunk, chunk)]

        def out_slice(ci):
            h, b = divmod(ci, B)
            return out_hbm.at[pl.ds(b * S + pos_base + h * chunk, chunk)]

        def pos_slice(h):
            return pos_hbm.at[pl.ds(pos_base + h * chunk, chunk)]

        ahead = nbuf - 2
        pltpu.async_copy(pos_slice(0), pos_v, psem)
        for b in range(B):
            idx_copy(b).start()
        for ci in range(ahead):
            idx_copy(ci).wait()
            pltpu.async_copy(tok_hbm.at[idx_slice(ci)], bufs[ci], gsem)
        for b in range(ahead, B):
            idx_copy(b).wait()

        for ci in range(n_chunks):
            h, b = divmod(ci, B)
            buf = bufs[ci % nbuf]
            pltpu.make_async_copy(tok_hbm.at[idx_slice(ci)], buf, gsem).wait()
            if ci + ahead < n_chunks:
                if ci >= 2:
                    pltpu.make_async_copy(
                        bufs[(ci + ahead) % nbuf], out_slice(ci - 2), wsem
                    ).wait()
                pltpu.async_copy(
                    tok_hbm.at[idx_slice(ci + ahead)],
                    bufs[(ci + ahead) % nbuf],
                    gsem,
                )
            if b == 0:
                pltpu.make_async_copy(pos_slice(h), pos_v, psem).wait()

            def add_row(r, _, buf=buf):
                for c in range(D // _L):
                    sl = pl.ds(c * _L, _L)
                    plsc.addupdate(buf.at[r, sl], pos_v[r, sl])
                return 0

            lax.fori_loop(0, chunk, add_row, 0)
            if b == B - 1 and h + 1 < halves:
                pltpu.async_copy(pos_slice(h + 1), pos_v, psem)
            pltpu.async_copy(buf, out_slice(ci), wsem)

        for ci in range(max(0, n_chunks - nbuf), n_chunks):
            pltpu.make_async_copy(bufs[ci % nbuf], out_slice(ci), wsem).wait()

    return embed


def kernel(inputs, token_table, pos_table):
    B, S = inputs.shape
    V, D = token_table.shape
    idx2d = inputs.astype(jnp.int32)
    embed = _make_embed_kernel(B, S, D, n_workers=32, chunk=32, nbuf=4)
    out = embed(idx2d, token_table, pos_table)
    return out.reshape(B, S, D)
